# baseline (speedup 1.0000x reference)
import numpy as np
import jax
import jax.numpy as jnp
from jax.experimental import pallas as pl
from jax.experimental.pallas import tpu as pltpu

_DILATIONS = [1, 2, 1, 2]
_HIS_LEN = 12


def _composed_avg_matrix():
    A = np.eye(_HIS_LEN, dtype=np.float64)
    size = _HIS_LEN
    for d in _DILATIONS:
        m = size - d
        L = np.zeros((m, size))
        for i in range(m):
            L[i, i] = 0.5
            L[i, i + d] = 0.5
        A = L @ A
        size = m
    return A.astype(np.float32)


_A = _composed_avg_matrix()
_T = _HIS_LEN
_OUT_LEN = 12


def _fused_kernel(m_ref, bias_ref, x_ref, w_in_ref, w_out2_ref, b_out2_ref,
                  out_ref):
    _, T, n2, f2 = x_ref.shape
    x = x_ref[0].reshape(T, n2 * f2)
    z = jnp.dot(m_ref[...], x, preferred_element_type=jnp.float32)
    z = z.reshape(_OUT_LEN, n2, f2)
    h = jax.lax.dot_general(z, w_in_ref[...], (((2,), (0,)), ((), ())),
                            preferred_element_type=jnp.float32)
    h = jnp.maximum(h + bias_ref[...][:, None, :], 0.0)
    y = jax.lax.dot_general(h, w_out2_ref[...], (((2,), (0,)), ((), ())),
                            preferred_element_type=jnp.float32)
    out_ref[0] = y + b_out2_ref[...][None]


def kernel(inputs, W_in, b_in, W_out1, b_out1, W_out2, b_out2):
    B, T, N, F = inputs.shape
    HID = W_in.shape[1]
    OUT_DIM = W_out2.shape[1]
    N2 = N // 2

    M = W_out1.T @ jnp.asarray(_A)
    bias = jnp.sum(M, axis=1, keepdims=True) * b_in[None, :] \
        + b_out1[:, None]
    bias2 = jnp.concatenate([bias, bias], axis=1)

    zf = jnp.zeros((F, HID), jnp.float32)
    w_in2 = jnp.block([[W_in, zf], [zf, W_in]])
    zh = jnp.zeros((HID, OUT_DIM), jnp.float32)
    w_out2b = jnp.block([[W_out2, zh], [zh, W_out2]])
    b_out2b = jnp.concatenate([b_out2, b_out2])[None, :]

    x2 = inputs.reshape(B, T, N2, 2 * F)

    out = pl.pallas_call(
        _fused_kernel,
        grid=(B,),
        in_specs=[
            pl.BlockSpec((_OUT_LEN, T), lambda b: (0, 0)),
            pl.BlockSpec((_OUT_LEN, 2 * HID), lambda b: (0, 0)),
            pl.BlockSpec((1, T, N2, 2 * F), lambda b: (b, 0, 0, 0)),
            pl.BlockSpec((2 * F, 2 * HID), lambda b: (0, 0)),
            pl.BlockSpec((2 * HID, 2 * OUT_DIM), lambda b: (0, 0)),
            pl.BlockSpec((1, 2 * OUT_DIM), lambda b: (0, 0)),
        ],
        out_specs=pl.BlockSpec((1, _OUT_LEN, N2, 2 * OUT_DIM),
                               lambda b: (b, 0, 0, 0)),
        out_shape=jax.ShapeDtypeStruct((B, _OUT_LEN, N2, 2 * OUT_DIM),
                                       jnp.float32),
        compiler_params=pltpu.CompilerParams(
            dimension_semantics=("parallel",)),
    )(M, bias2, x2, w_in2, w_out2b, b_out2b)
    return out.reshape(B, _OUT_LEN, N, OUT_DIM)


# speedup vs baseline: 1.0123x; 1.0123x over previous
import numpy as np
import jax
import jax.numpy as jnp
from jax.experimental import pallas as pl
from jax.experimental.pallas import tpu as pltpu

_DILATIONS = [1, 2, 1, 2]
_HIS_LEN = 12


def _composed_avg_matrix():
    A = np.eye(_HIS_LEN, dtype=np.float64)
    size = _HIS_LEN
    for d in _DILATIONS:
        m = size - d
        L = np.zeros((m, size))
        for i in range(m):
            L[i, i] = 0.5
            L[i, i + d] = 0.5
        A = L @ A
        size = m
    return A.astype(np.float32)


_A = _composed_avg_matrix()
_T = _HIS_LEN
_OUT_LEN = 12


def _fused_kernel(m_ref, bias_ref, x_ref, w_in_ref, w_out2_ref, b_out2_ref,
                  out_ref):
    x = x_ref[0]
    w_in = w_in_ref[...]
    w_out2 = w_out2_ref[...]
    b_out2 = b_out2_ref[...]
    for o in range(0, _OUT_LEN, 2):
        xt = x[0]
        z0 = m_ref[o, 0] * xt
        z1 = m_ref[o + 1, 0] * xt
        for t in range(1, _T):
            xt = x[t]
            z0 = z0 + m_ref[o, t] * xt
            z1 = z1 + m_ref[o + 1, t] * xt
        for j, z in ((0, z0), (1, z1)):
            h = jnp.dot(z, w_in, preferred_element_type=jnp.float32)
            h = jnp.maximum(h + bias_ref[o + j], 0.0)
            y = jnp.dot(h, w_out2, preferred_element_type=jnp.float32)
            out_ref[0, o + j] = y + b_out2


def kernel(inputs, W_in, b_in, W_out1, b_out1, W_out2, b_out2):
    B, T, N, F = inputs.shape
    HID = W_in.shape[1]
    OUT_DIM = W_out2.shape[1]
    N2 = N // 2

    M = W_out1.T @ jnp.asarray(_A)
    bias = jnp.sum(M, axis=1, keepdims=True) * b_in[None, :] \
        + b_out1[:, None]
    bias2 = jnp.concatenate([bias, bias], axis=1)

    zf = jnp.zeros((F, HID), jnp.float32)
    w_in2 = jnp.block([[W_in, zf], [zf, W_in]])
    zh = jnp.zeros((HID, OUT_DIM), jnp.float32)
    w_out2b = jnp.block([[W_out2, zh], [zh, W_out2]])
    b_out2b = jnp.concatenate([b_out2, b_out2])[None, :]

    x2 = inputs.reshape(B, T, N2, 2 * F)

    out = pl.pallas_call(
        _fused_kernel,
        grid=(B,),
        in_specs=[
            pl.BlockSpec(memory_space=pltpu.SMEM),
            pl.BlockSpec((_OUT_LEN, 2 * HID), lambda b: (0, 0)),
            pl.BlockSpec((1, T, N2, 2 * F), lambda b: (b, 0, 0, 0)),
            pl.BlockSpec((2 * F, 2 * HID), lambda b: (0, 0)),
            pl.BlockSpec((2 * HID, 2 * OUT_DIM), lambda b: (0, 0)),
            pl.BlockSpec((1, 2 * OUT_DIM), lambda b: (0, 0)),
        ],
        out_specs=pl.BlockSpec((1, _OUT_LEN, N2, 2 * OUT_DIM),
                               lambda b: (b, 0, 0, 0)),
        out_shape=jax.ShapeDtypeStruct((B, _OUT_LEN, N2, 2 * OUT_DIM),
                                       jnp.float32),
        compiler_params=pltpu.CompilerParams(
            dimension_semantics=("parallel",)),
    )(M, bias2, x2, w_in2, w_out2b, b_out2b)
    return out.reshape(B, _OUT_LEN, N, OUT_DIM)
